# Initial kernel scaffold; baseline (speedup 1.0000x reference)
import jax
import jax.numpy as jnp
from jax.experimental import pallas as pl


def kernel(x_sparse, x_dense, emb_tables, lin_tables, W0, b0, W1, b1, Wo, bo):
    raise NotImplementedError("write your pallas kernel here")


# speedup vs baseline: 1.4301x; 1.4301x over previous
import functools

import jax
import jax.numpy as jnp
import numpy as np
from jax import lax
from jax.experimental import pallas as pl
from jax.experimental.pallas import tpu as pltpu
from jax.experimental.pallas import tpu_sc as plsc

_B = 16384
_F = 26
_V = 100000
_D = 16

_NC = 2
_NS = 16
_NW = _NC * _NS
_N = _B * _F
_PER_W = _N // _NW
_CHUNK = 128
_NBUF = 2
_NSTEPS = _PER_W // (_CHUNK * _NBUF)


def _gather_body(emb_hbm, lin16_hbm, idx_hbm, emb_out, lin_out,
                 idx_bufs, idx16_bufs, row_bufs, lin16_bufs, linval_bufs,
                 sems_e, sems_l):
    wid = lax.axis_index("s") * _NC + lax.axis_index("c")
    base = wid * _PER_W
    lane_iota = lax.iota(jnp.int32, 16)

    def step(m, carry):
        offs = [base + (m * _NBUF + b) * _CHUNK for b in range(_NBUF)]
        for b in range(_NBUF):
            pltpu.sync_copy(idx_hbm.at[pl.ds(offs[b], _CHUNK)], idx_bufs[b])
            for g in range(_CHUNK // 16):
                sl = pl.ds(g * 16, 16)
                idx16_bufs[b][sl] = jnp.right_shift(idx_bufs[b][sl], 4)
        cps = []
        for b in range(_NBUF):
            cps.append(pltpu.async_copy(emb_hbm.at[idx_bufs[b]],
                                        row_bufs[b], sems_e[b]))
            cps.append(pltpu.async_copy(lin16_hbm.at[idx16_bufs[b]],
                                        lin16_bufs[b], sems_l[b]))
        for c in cps:
            c.wait()
        for b in range(_NBUF):
            for g in range(_CHUNK // 16):
                sl = pl.ds(g * 16, 16)
                rows = lane_iota + g * 16
                lanes = jnp.bitwise_and(idx_bufs[b][sl], 15)
                linval_bufs[b][sl] = plsc.load_gather(
                    lin16_bufs[b], [rows, lanes])
            pltpu.sync_copy(row_bufs[b], emb_out.at[pl.ds(offs[b], _CHUNK)])
            pltpu.sync_copy(linval_bufs[b], lin_out.at[pl.ds(offs[b], _CHUNK)])
        return carry

    lax.fori_loop(0, _NSTEPS, step, 0)


@functools.lru_cache(maxsize=None)
def _make_sc_gather():
    return functools.partial(
        pl.kernel,
        mesh=plsc.VectorSubcoreMesh(core_axis_name="c", subcore_axis_name="s",
                                    num_cores=_NC, num_subcores=_NS),
        out_type=[
            jax.ShapeDtypeStruct((_N, _D), jnp.float32),
            jax.ShapeDtypeStruct((_N,), jnp.float32),
        ],
        scratch_types=[
            [pltpu.VMEM((_CHUNK,), jnp.int32) for _ in range(_NBUF)],
            [pltpu.VMEM((_CHUNK,), jnp.int32) for _ in range(_NBUF)],
            [pltpu.VMEM((_CHUNK, _D), jnp.float32) for _ in range(_NBUF)],
            [pltpu.VMEM((_CHUNK, 16), jnp.float32) for _ in range(_NBUF)],
            [pltpu.VMEM((_CHUNK,), jnp.float32) for _ in range(_NBUF)],
            [pltpu.SemaphoreType.DMA for _ in range(_NBUF)],
            [pltpu.SemaphoreType.DMA for _ in range(_NBUF)],
        ],
        compiler_params=pltpu.CompilerParams(use_tc_tiling_on_sc=False,
                                             needs_layout_passes=False),
    )(_gather_body)


def _mlp_body(emb_ref, xd_ref, lin_ref, w0e_ref, w0d_ref, b0_ref,
              w1_ref, b1_ref, wo_ref, bo_ref, s_ref, out_ref):
    dot = functools.partial(jnp.dot, preferred_element_type=jnp.float32,
                            precision=lax.Precision.HIGHEST)
    emb = emb_ref[...]
    h = dot(emb, w0e_ref[...])
    h = h + dot(xd_ref[...], w0d_ref[...])
    h = jnp.maximum(h + b0_ref[...], 0.0)
    h = jnp.maximum(dot(h, w1_ref[...]) + b1_ref[...], 0.0)
    dnn = dot(h, wo_ref[...]) + bo_ref[...]
    s = dot(emb, s_ref[...])
    fm = 0.5 * (jnp.sum(s * s, axis=1, keepdims=True)
                - jnp.sum(emb * emb, axis=1, keepdims=True))
    lin_logit = jnp.sum(lin_ref[...], axis=1, keepdims=True)
    logit = dnn + fm + lin_logit
    out_ref[...] = 1.0 / (1.0 + jnp.exp(-logit))


def _tc_mlp(emb2, x_dense, lin2, w0e, w0d, b0, w1, b1, wo, bo, smat):
    bm = 1024
    grid = (_B // bm,)
    return pl.pallas_call(
        _mlp_body,
        grid=grid,
        in_specs=[
            pl.BlockSpec((bm, _F * _D), lambda i: (i, 0)),
            pl.BlockSpec((bm, x_dense.shape[1]), lambda i: (i, 0)),
            pl.BlockSpec((bm, _F), lambda i: (i, 0)),
            pl.BlockSpec(w0e.shape, lambda i: (0, 0)),
            pl.BlockSpec(w0d.shape, lambda i: (0, 0)),
            pl.BlockSpec(b0.shape, lambda i: (0,)),
            pl.BlockSpec(w1.shape, lambda i: (0, 0)),
            pl.BlockSpec(b1.shape, lambda i: (0,)),
            pl.BlockSpec(wo.shape, lambda i: (0, 0)),
            pl.BlockSpec(bo.shape, lambda i: (0,)),
            pl.BlockSpec(smat.shape, lambda i: (0, 0)),
        ],
        out_specs=pl.BlockSpec((bm, 1), lambda i: (i, 0)),
        out_shape=jax.ShapeDtypeStruct((_B, 1), jnp.float32),
    )(emb2, x_dense, lin2, w0e, w0d, b0, w1, b1, wo, bo, smat)


def kernel(x_sparse, x_dense, emb_tables, lin_tables, W0, b0, W1, b1, Wo, bo):
    nf = emb_tables.shape[0]
    v = emb_tables.shape[1]
    d = emb_tables.shape[2]
    idx = (x_sparse.astype(jnp.int32)
           + jnp.arange(nf, dtype=jnp.int32)[None, :] * v).reshape(-1)
    emb_flat = emb_tables.reshape(nf * v, d)
    lin16 = lin_tables.reshape(nf * v // 16, 16)
    emb_rows, lin_vals = _make_sc_gather()(emb_flat, lin16, idx)
    emb2 = emb_rows.reshape(_B, nf * d)
    lin2 = lin_vals.reshape(_B, nf)
    w0e = W0[:nf * d]
    w0d = W0[nf * d:]
    smat = jnp.tile(jnp.eye(d, dtype=jnp.float32), (nf, 1))
    out = _tc_mlp(emb2, x_dense, lin2, w0e, w0d, b0, W1, b1, Wo, bo, smat)
    return out.reshape(_B)
